# baseline (speedup 1.0000x reference)
import functools

import jax
import jax.numpy as jnp
from jax import lax
from jax.experimental import pallas as pl
from jax.experimental.pallas import tpu as pltpu
from jax.experimental.pallas import tpu_sc as plsc

B, N, D, K = 4, 4096, 29, 32
C_IN = D + 3
C_OUT = 64
R = B * N * K


S1 = 512


_CW = 64
_CH1 = N // _CW
_M1 = 10
_NCAND = _CH1 * _M1


def _knn_body(xyz_ref, p_ref, idx_ref, cv_ref, ci_ref):
    b = pl.program_id(0)
    j = pl.program_id(1)
    pfull = p_ref[0]
    tile = xyz_ref[0, :, pl.ds(j * S1, S1)]
    nall = jnp.sum(pfull * pfull, axis=1, keepdims=True)
    nrow = jnp.sum(tile * tile, axis=0, keepdims=True)
    dot = lax.dot_general(pfull, tile, (((1,), (0,)), ((), ())),
                          preferred_element_type=jnp.float32)
    d2 = jnp.reshape(nall + nrow - 2.0 * dot, (_CH1, _CW, S1))
    gio = (lax.broadcasted_iota(jnp.int32, (_CH1, _CW, S1), 0) * _CW
           + lax.broadcasted_iota(jnp.int32, (_CH1, _CW, S1), 1))
    big_i = jnp.int32(2**30)
    big_f = jnp.float32(jnp.inf)
    for m1 in range(_M1):
        mv = jnp.min(d2, axis=1, keepdims=True)
        sel = jnp.min(jnp.where(d2 == mv, gio, big_i),
                      axis=1, keepdims=True)
        cv_ref[pl.ds(m1 * _CH1, _CH1), :] = mv[:, 0, :]
        ci_ref[pl.ds(m1 * _CH1, _CH1), :] = sel[:, 0, :]
        d2 = jnp.where(gio == sel, big_f, d2)
    cands = cv_ref[...]
    tidx = ci_ref[...]
    row = lax.broadcasted_iota(jnp.int32, (K, S1), 0)
    acc_t = jnp.zeros((K, S1), jnp.int32)
    for k in range(K):
        m = jnp.min(cands, axis=0, keepdims=True)
        sel = jnp.min(jnp.where(cands == m, tidx, big_i),
                      axis=0, keepdims=True)
        acc_t = jnp.where(row == k, sel, acc_t)
        cands = jnp.where(tidx == sel, big_f, cands)
    idx_ref[0] = jnp.transpose(acc_t) + b * N


def _knn(xyz, p):
    return pl.pallas_call(
        _knn_body,
        grid=(B, N // S1),
        in_specs=[
            pl.BlockSpec((1, 3, N), lambda b, j: (b, 0, 0)),
            pl.BlockSpec((1, N, 3), lambda b, j: (b, 0, 0)),
        ],
        out_specs=pl.BlockSpec((1, S1, K), lambda b, j: (b, j, 0)),
        out_shape=jax.ShapeDtypeStruct((B, N, K), jnp.int32),
        scratch_shapes=[
            pltpu.VMEM((_NCAND, S1), jnp.float32),
            pltpu.VMEM((_NCAND, S1), jnp.int32),
        ],
    )(xyz, p)


_NC, _NS = 2, 16
_NW = _NC * _NS
_RW = R // _NW
_CG = 128
_NCHUNK = _RW // _CG


def _gather_sc(table, idx_flat):
    mesh = plsc.VectorSubcoreMesh(core_axis_name="c", subcore_axis_name="s")

    @functools.partial(
        pl.kernel,
        mesh=mesh,
        out_type=jax.ShapeDtypeStruct((R, C_IN), jnp.float32),
        compiler_params=pltpu.CompilerParams(use_tc_tiling_on_sc=False),
        scratch_types=[
            pltpu.VMEM((_RW,), jnp.int32),
            pltpu.VMEM((2, _CG, C_IN), jnp.float32),
            pltpu.SemaphoreType.DMA,
        ],
    )
    def k(table_hbm, idx_hbm, out_hbm, idx_v, buf_v, sem):
        wid = lax.axis_index("s") * _NC + lax.axis_index("c")
        base = wid * _RW
        pltpu.sync_copy(idx_hbm.at[pl.ds(base, _RW)], idx_v)

        def body(i, carry):
            cp = pltpu.async_copy(
                table_hbm.at[idx_v.at[pl.ds(i * _CG, _CG)]],
                buf_v.at[0], sem)
            cp.wait()
            pltpu.sync_copy(buf_v.at[0],
                            out_hbm.at[pl.ds(base + i * _CG, _CG)])
            return carry

        lax.fori_loop(0, _NCHUNK, body, 0)

    return k(table, idx_flat)


S3 = 512
T3 = S3 * K
_TPB = N // S3


def _mlp_body(g_ref, p_ref, w1_ref, b1_ref, w2_ref, b2_ref, w3_ref, b3_ref,
              out_ref):
    x = g_ref[...]
    a = p_ref[0]
    w1x = w1_ref[:, D:C_IN]
    corr = lax.dot_general(a, w1x, (((1,), (1,)), ((), ())),
                           preferred_element_type=jnp.float32)
    corr = jnp.reshape(
        jnp.broadcast_to(corr[:, None, :], (S3, K, C_IN)), (T3, C_IN))
    h = lax.dot_general(x, w1_ref[...], (((1,), (1,)), ((), ())),
                        preferred_element_type=jnp.float32)
    h = jnp.maximum(h - corr + b1_ref[...], 0.0)
    h = lax.dot_general(h, w2_ref[...], (((1,), (1,)), ((), ())),
                        preferred_element_type=jnp.float32)
    h = jnp.maximum(h + b2_ref[...], 0.0)
    y = lax.dot_general(w3_ref[...], h, (((1,), (1,)), ((), ())),
                        preferred_element_type=jnp.float32)
    out_ref[0] = y + b3_ref[...]


def _mlp(g, p, W1, b1, W2, b2, W3, b3):
    full = lambda shape: pl.BlockSpec(shape, lambda t: tuple(0 for _ in shape))
    out = pl.pallas_call(
        _mlp_body,
        grid=(B * N // S3,),
        in_specs=[
            pl.BlockSpec((T3, C_IN), lambda t: (t, 0)),
            pl.BlockSpec((1, S3, 3), lambda t: (t // _TPB, t % _TPB, 0)),
            full((C_IN, C_IN)),
            full((1, C_IN)),
            full((C_IN, C_IN)),
            full((1, C_IN)),
            full((C_OUT, C_IN)),
            full((C_OUT, 1)),
        ],
        out_specs=pl.BlockSpec((1, C_OUT, T3), lambda t: (t // _TPB, 0, t % _TPB)),
        out_shape=jax.ShapeDtypeStruct((B, C_OUT, N * K), jnp.float32),
    )(g, p, W1, b1.reshape(1, C_IN), W2, b2.reshape(1, C_IN), W3,
      b3.reshape(C_OUT, 1))
    return out.reshape(B, C_OUT, N, K)


def kernel(xyz, points, W1, b1, W2, b2, W3, b3):
    p = jnp.moveaxis(xyz, 1, 2)
    table = jnp.concatenate(
        [jnp.moveaxis(points, 1, 2), p], axis=-1).reshape(B * N, C_IN)
    idx = _knn(xyz, p)
    g = _gather_sc(table, idx.reshape(R))
    return _mlp(g, p, W1, b1, W2, b2, W3, b3)


# speedup vs baseline: 1.0227x; 1.0227x over previous
import functools

import jax
import jax.numpy as jnp
from jax import lax
from jax.experimental import pallas as pl
from jax.experimental.pallas import tpu as pltpu
from jax.experimental.pallas import tpu_sc as plsc

B, N, D, K = 4, 4096, 29, 32
C_IN = D + 3
C_OUT = 64
R = B * N * K


S1 = 128


_CW = 64
_CH1 = N // _CW
_M1 = 10
_NCAND = _CH1 * _M1


def _knn_body(xyz_ref, p_ref, idx_ref, cv_ref, ci_ref):
    b = pl.program_id(0)
    j = pl.program_id(1)
    pfull = p_ref[0]
    tile = xyz_ref[0, :, pl.ds(j * S1, S1)]
    nall = jnp.sum(pfull * pfull, axis=1, keepdims=True)
    nrow = jnp.sum(tile * tile, axis=0, keepdims=True)
    dot = lax.dot_general(pfull, tile, (((1,), (0,)), ((), ())),
                          preferred_element_type=jnp.float32)
    d2 = jnp.reshape(nall + nrow - 2.0 * dot, (_CH1, _CW, S1))
    gio = (lax.broadcasted_iota(jnp.int32, (_CH1, _CW, S1), 0) * _CW
           + lax.broadcasted_iota(jnp.int32, (_CH1, _CW, S1), 1))
    big_i = jnp.int32(2**30)
    big_f = jnp.float32(jnp.inf)
    for m1 in range(_M1):
        mv = jnp.min(d2, axis=1, keepdims=True)
        sel = jnp.min(jnp.where(d2 == mv, gio, big_i),
                      axis=1, keepdims=True)
        cv_ref[pl.ds(m1 * _CH1, _CH1), :] = mv[:, 0, :]
        ci_ref[pl.ds(m1 * _CH1, _CH1), :] = sel[:, 0, :]
        d2 = jnp.where(gio == sel, big_f, d2)
    cands = cv_ref[...]
    tidx = ci_ref[...]
    row = lax.broadcasted_iota(jnp.int32, (K, S1), 0)
    acc_t = jnp.zeros((K, S1), jnp.int32)
    for k in range(K):
        m = jnp.min(cands, axis=0, keepdims=True)
        sel = jnp.min(jnp.where(cands == m, tidx, big_i),
                      axis=0, keepdims=True)
        acc_t = jnp.where(row == k, sel, acc_t)
        cands = jnp.where(tidx == sel, big_f, cands)
    idx_ref[0] = jnp.transpose(acc_t) + b * N


def _knn(xyz, p):
    return pl.pallas_call(
        _knn_body,
        grid=(B, N // S1),
        in_specs=[
            pl.BlockSpec((1, 3, N), lambda b, j: (b, 0, 0)),
            pl.BlockSpec((1, N, 3), lambda b, j: (b, 0, 0)),
        ],
        out_specs=pl.BlockSpec((1, S1, K), lambda b, j: (b, j, 0)),
        out_shape=jax.ShapeDtypeStruct((B, N, K), jnp.int32),
        scratch_shapes=[
            pltpu.VMEM((_NCAND, S1), jnp.float32),
            pltpu.VMEM((_NCAND, S1), jnp.int32),
        ],
    )(xyz, p)


_NC, _NS = 2, 16
_NW = _NC * _NS
_RW = R // _NW
_CG = 128
_NCHUNK = _RW // _CG


def _gather_sc(table, idx_flat):
    mesh = plsc.VectorSubcoreMesh(core_axis_name="c", subcore_axis_name="s")

    @functools.partial(
        pl.kernel,
        mesh=mesh,
        out_type=jax.ShapeDtypeStruct((R, C_IN), jnp.float32),
        compiler_params=pltpu.CompilerParams(use_tc_tiling_on_sc=False),
        scratch_types=[
            pltpu.VMEM((_RW,), jnp.int32),
            pltpu.VMEM((2, _CG, C_IN), jnp.float32),
            pltpu.SemaphoreType.DMA,
        ],
    )
    def k(table_hbm, idx_hbm, out_hbm, idx_v, buf_v, sem):
        wid = lax.axis_index("s") * _NC + lax.axis_index("c")
        base = wid * _RW
        pltpu.sync_copy(idx_hbm.at[pl.ds(base, _RW)], idx_v)

        def body(i, carry):
            cp = pltpu.async_copy(
                table_hbm.at[idx_v.at[pl.ds(i * _CG, _CG)]],
                buf_v.at[0], sem)
            cp.wait()
            pltpu.sync_copy(buf_v.at[0],
                            out_hbm.at[pl.ds(base + i * _CG, _CG)])
            return carry

        lax.fori_loop(0, _NCHUNK, body, 0)

    return k(table, idx_flat)


S3 = 512
T3 = S3 * K
_TPB = N // S3


def _mlp_body(g_ref, p_ref, w1_ref, b1_ref, w2_ref, b2_ref, w3_ref, b3_ref,
              out_ref):
    x = g_ref[...]
    a = p_ref[0]
    w1x = w1_ref[:, D:C_IN]
    corr = lax.dot_general(a, w1x, (((1,), (1,)), ((), ())),
                           preferred_element_type=jnp.float32)
    corr = jnp.reshape(
        jnp.broadcast_to(corr[:, None, :], (S3, K, C_IN)), (T3, C_IN))
    h = lax.dot_general(x, w1_ref[...], (((1,), (1,)), ((), ())),
                        preferred_element_type=jnp.float32)
    h = jnp.maximum(h - corr + b1_ref[...], 0.0)
    h = lax.dot_general(h, w2_ref[...], (((1,), (1,)), ((), ())),
                        preferred_element_type=jnp.float32)
    h = jnp.maximum(h + b2_ref[...], 0.0)
    y = lax.dot_general(w3_ref[...], h, (((1,), (1,)), ((), ())),
                        preferred_element_type=jnp.float32)
    out_ref[0] = y + b3_ref[...]


def _mlp(g, p, W1, b1, W2, b2, W3, b3):
    full = lambda shape: pl.BlockSpec(shape, lambda t: tuple(0 for _ in shape))
    out = pl.pallas_call(
        _mlp_body,
        grid=(B * N // S3,),
        in_specs=[
            pl.BlockSpec((T3, C_IN), lambda t: (t, 0)),
            pl.BlockSpec((1, S3, 3), lambda t: (t // _TPB, t % _TPB, 0)),
            full((C_IN, C_IN)),
            full((1, C_IN)),
            full((C_IN, C_IN)),
            full((1, C_IN)),
            full((C_OUT, C_IN)),
            full((C_OUT, 1)),
        ],
        out_specs=pl.BlockSpec((1, C_OUT, T3), lambda t: (t // _TPB, 0, t % _TPB)),
        out_shape=jax.ShapeDtypeStruct((B, C_OUT, N * K), jnp.float32),
    )(g, p, W1, b1.reshape(1, C_IN), W2, b2.reshape(1, C_IN), W3,
      b3.reshape(C_OUT, 1))
    return out.reshape(B, C_OUT, N, K)


def kernel(xyz, points, W1, b1, W2, b2, W3, b3):
    p = jnp.moveaxis(xyz, 1, 2)
    table = jnp.concatenate(
        [jnp.moveaxis(points, 1, 2), p], axis=-1).reshape(B * N, C_IN)
    idx = _knn(xyz, p)
    g = _gather_sc(table, idx.reshape(R))
    return _mlp(g, p, W1, b1, W2, b2, W3, b3)


# speedup vs baseline: 1.0315x; 1.0086x over previous
import functools

import jax
import jax.numpy as jnp
from jax import lax
from jax.experimental import pallas as pl
from jax.experimental.pallas import tpu as pltpu
from jax.experimental.pallas import tpu_sc as plsc

B, N, D, K = 4, 4096, 29, 32
C_IN = D + 3
C_OUT = 64
R = B * N * K


S1 = 256


_CW = 128
_CH1 = N // _CW
_M1 = 12
_MP = 16
_NCAND = _CH1 * _MP


def _knn_body(xyz_ref, p_ref, idx_ref, d2_ref, cv_ref, ci_ref):
    b = pl.program_id(0)
    j = pl.program_id(1)
    pfull = p_ref[0]
    tile = xyz_ref[0, :, pl.ds(j * S1, S1)]
    nall = jnp.sum(pfull * pfull, axis=1, keepdims=True)
    nrow = jnp.sum(tile * tile, axis=0, keepdims=True)
    dot = lax.dot_general(pfull, tile, (((1,), (0,)), ((), ())),
                          preferred_element_type=jnp.float32)
    d2_ref[...] = jnp.reshape(nall + nrow - 2.0 * dot, (_CH1, _CW, S1))
    big_i = jnp.int32(2**30)
    big_f = jnp.float32(jnp.inf)
    rio = lax.broadcasted_iota(jnp.int32, (_CW, S1), 0)
    rio16 = lax.broadcasted_iota(jnp.int32, (_MP, S1), 0)

    def chunk_body(c, carry):
        sub = d2_ref[c]
        grio = rio + c * _CW
        mvacc = jnp.full((_MP, S1), big_f, jnp.float32)
        siacc = jnp.full((_MP, S1), big_i, jnp.int32)
        for m1 in range(_M1):
            m = jnp.min(sub, axis=0, keepdims=True)
            sel = jnp.min(jnp.where(sub == m, grio, big_i),
                          axis=0, keepdims=True)
            mvacc = jnp.where(rio16 == m1, m, mvacc)
            siacc = jnp.where(rio16 == m1, sel, siacc)
            sub = jnp.where(grio == sel, big_f, sub)
        cv_ref[c] = mvacc
        ci_ref[c] = siacc
        return carry

    lax.fori_loop(0, _CH1, chunk_body, 0)

    cands = jnp.reshape(cv_ref[...], (_NCAND, S1))
    tidx = jnp.reshape(ci_ref[...], (_NCAND, S1))
    row = lax.broadcasted_iota(jnp.int32, (K, S1), 0)
    acc_t = jnp.zeros((K, S1), jnp.int32)
    for k in range(K):
        m = jnp.min(cands, axis=0, keepdims=True)
        sel = jnp.min(jnp.where(cands == m, tidx, big_i),
                      axis=0, keepdims=True)
        acc_t = jnp.where(row == k, sel, acc_t)
        cands = jnp.where(tidx == sel, big_f, cands)
    idx_ref[0] = jnp.transpose(acc_t) + b * N


def _knn(xyz, p):
    return pl.pallas_call(
        _knn_body,
        grid=(B, N // S1),
        in_specs=[
            pl.BlockSpec((1, 3, N), lambda b, j: (b, 0, 0)),
            pl.BlockSpec((1, N, 3), lambda b, j: (b, 0, 0)),
        ],
        out_specs=pl.BlockSpec((1, S1, K), lambda b, j: (b, j, 0)),
        out_shape=jax.ShapeDtypeStruct((B, N, K), jnp.int32),
        scratch_shapes=[
            pltpu.VMEM((_CH1, _CW, S1), jnp.float32),
            pltpu.VMEM((_CH1, _MP, S1), jnp.float32),
            pltpu.VMEM((_CH1, _MP, S1), jnp.int32),
        ],
    )(xyz, p)


_NC, _NS = 2, 16
_NW = _NC * _NS
_RW = R // _NW
_CG = 128
_NCHUNK = _RW // _CG


def _gather_sc(table, idx_flat):
    mesh = plsc.VectorSubcoreMesh(core_axis_name="c", subcore_axis_name="s")

    @functools.partial(
        pl.kernel,
        mesh=mesh,
        out_type=jax.ShapeDtypeStruct((R, C_IN), jnp.float32),
        compiler_params=pltpu.CompilerParams(use_tc_tiling_on_sc=False),
        scratch_types=[
            pltpu.VMEM((_RW,), jnp.int32),
            pltpu.VMEM((2, _CG, C_IN), jnp.float32),
            pltpu.SemaphoreType.DMA,
        ],
    )
    def k(table_hbm, idx_hbm, out_hbm, idx_v, buf_v, sem):
        wid = lax.axis_index("s") * _NC + lax.axis_index("c")
        base = wid * _RW
        pltpu.sync_copy(idx_hbm.at[pl.ds(base, _RW)], idx_v)

        def body(i, carry):
            cp = pltpu.async_copy(
                table_hbm.at[idx_v.at[pl.ds(i * _CG, _CG)]],
                buf_v.at[0], sem)
            cp.wait()
            pltpu.sync_copy(buf_v.at[0],
                            out_hbm.at[pl.ds(base + i * _CG, _CG)])
            return carry

        lax.fori_loop(0, _NCHUNK, body, 0)

    return k(table, idx_flat)


S3 = 512
T3 = S3 * K
_TPB = N // S3


def _mlp_body(g_ref, p_ref, w1_ref, b1_ref, w2_ref, b2_ref, w3_ref, b3_ref,
              out_ref):
    x = g_ref[...]
    a = p_ref[0]
    w1x = w1_ref[:, D:C_IN]
    corr = lax.dot_general(a, w1x, (((1,), (1,)), ((), ())),
                           preferred_element_type=jnp.float32)
    corr = jnp.reshape(
        jnp.broadcast_to(corr[:, None, :], (S3, K, C_IN)), (T3, C_IN))
    h = lax.dot_general(x, w1_ref[...], (((1,), (1,)), ((), ())),
                        preferred_element_type=jnp.float32)
    h = jnp.maximum(h - corr + b1_ref[...], 0.0)
    h = lax.dot_general(h, w2_ref[...], (((1,), (1,)), ((), ())),
                        preferred_element_type=jnp.float32)
    h = jnp.maximum(h + b2_ref[...], 0.0)
    y = lax.dot_general(w3_ref[...], h, (((1,), (1,)), ((), ())),
                        preferred_element_type=jnp.float32)
    out_ref[0] = y + b3_ref[...]


def _mlp(g, p, W1, b1, W2, b2, W3, b3):
    full = lambda shape: pl.BlockSpec(shape, lambda t: tuple(0 for _ in shape))
    out = pl.pallas_call(
        _mlp_body,
        grid=(B * N // S3,),
        in_specs=[
            pl.BlockSpec((T3, C_IN), lambda t: (t, 0)),
            pl.BlockSpec((1, S3, 3), lambda t: (t // _TPB, t % _TPB, 0)),
            full((C_IN, C_IN)),
            full((1, C_IN)),
            full((C_IN, C_IN)),
            full((1, C_IN)),
            full((C_OUT, C_IN)),
            full((C_OUT, 1)),
        ],
        out_specs=pl.BlockSpec((1, C_OUT, T3), lambda t: (t // _TPB, 0, t % _TPB)),
        out_shape=jax.ShapeDtypeStruct((B, C_OUT, N * K), jnp.float32),
    )(g, p, W1, b1.reshape(1, C_IN), W2, b2.reshape(1, C_IN), W3,
      b3.reshape(C_OUT, 1))
    return out.reshape(B, C_OUT, N, K)


def kernel(xyz, points, W1, b1, W2, b2, W3, b3):
    p = jnp.moveaxis(xyz, 1, 2)
    table = jnp.concatenate(
        [jnp.moveaxis(points, 1, 2), p], axis=-1).reshape(B * N, C_IN)
    idx = _knn(xyz, p)
    g = _gather_sc(table, idx.reshape(R))
    return _mlp(g, p, W1, b1, W2, b2, W3, b3)


# speedup vs baseline: 1.1636x; 1.1281x over previous
import functools

import jax
import jax.numpy as jnp
from jax import lax
from jax.experimental import pallas as pl
from jax.experimental.pallas import tpu as pltpu
from jax.experimental.pallas import tpu_sc as plsc

B, N, D, K = 4, 4096, 29, 32
C_IN = D + 3
C_OUT = 64
R = B * N * K


S1 = 256


_CW = 64
_CH1 = N // _CW
_M1 = 10
_NCAND = _CH1 * _M1


def _knn_body(xyz_ref, p_ref, idx_ref, cv_ref, ci_ref):
    b = pl.program_id(0)
    j = pl.program_id(1)
    pfull = p_ref[0]
    tile = xyz_ref[0, :, pl.ds(j * S1, S1)]
    nall = jnp.sum(pfull * pfull, axis=1, keepdims=True)
    nrow = jnp.sum(tile * tile, axis=0, keepdims=True)
    dot = lax.dot_general(pfull, tile, (((1,), (0,)), ((), ())),
                          preferred_element_type=jnp.float32)
    d2 = jnp.reshape(nall + nrow - 2.0 * dot, (_CH1, _CW, S1))
    gio = (lax.broadcasted_iota(jnp.int32, (_CH1, _CW, S1), 0) * _CW
           + lax.broadcasted_iota(jnp.int32, (_CH1, _CW, S1), 1))
    big_i = jnp.int32(2**30)
    big_f = jnp.float32(jnp.inf)
    for m1 in range(_M1):
        mv = jnp.min(d2, axis=1, keepdims=True)
        sel = jnp.min(jnp.where(d2 == mv, gio, big_i),
                      axis=1, keepdims=True)
        cv_ref[pl.ds(m1 * _CH1, _CH1), :] = mv[:, 0, :]
        ci_ref[pl.ds(m1 * _CH1, _CH1), :] = sel[:, 0, :]
        d2 = jnp.where(gio == sel, big_f, d2)
    cands = cv_ref[...]
    tidx = ci_ref[...]
    row = lax.broadcasted_iota(jnp.int32, (K, S1), 0)
    acc_t = jnp.zeros((K, S1), jnp.int32)
    for k in range(K):
        m = jnp.min(cands, axis=0, keepdims=True)
        sel = jnp.min(jnp.where(cands == m, tidx, big_i),
                      axis=0, keepdims=True)
        acc_t = jnp.where(row == k, sel, acc_t)
        cands = jnp.where(tidx == sel, big_f, cands)
    idx_ref[0] = jnp.transpose(acc_t) + b * N


def _knn(xyz, p):
    return pl.pallas_call(
        _knn_body,
        grid=(B, N // S1),
        in_specs=[
            pl.BlockSpec((1, 3, N), lambda b, j: (b, 0, 0)),
            pl.BlockSpec((1, N, 3), lambda b, j: (b, 0, 0)),
        ],
        out_specs=pl.BlockSpec((1, S1, K), lambda b, j: (b, j, 0)),
        out_shape=jax.ShapeDtypeStruct((B, N, K), jnp.int32),
        scratch_shapes=[
            pltpu.VMEM((_NCAND, S1), jnp.float32),
            pltpu.VMEM((_NCAND, S1), jnp.int32),
        ],
    )(xyz, p)


_NC, _NS = 2, 16
_NW = _NC * _NS
_RW = R // _NW
_CG = 128
_NCHUNK = _RW // _CG


def _gather_sc(table, idx_flat):
    mesh = plsc.VectorSubcoreMesh(core_axis_name="c", subcore_axis_name="s")

    @functools.partial(
        pl.kernel,
        mesh=mesh,
        out_type=jax.ShapeDtypeStruct((R, C_IN), jnp.float32),
        compiler_params=pltpu.CompilerParams(use_tc_tiling_on_sc=False),
        scratch_types=[
            pltpu.VMEM((_RW,), jnp.int32),
            pltpu.VMEM((2, _CG, C_IN), jnp.float32),
            pltpu.SemaphoreType.DMA,
            pltpu.SemaphoreType.DMA,
        ],
    )
    def k(table_hbm, idx_hbm, out_hbm, idx_v, buf_v, sem0, sem1):
        wid = lax.axis_index("s") * _NC + lax.axis_index("c")
        base = wid * _RW
        pltpu.sync_copy(idx_hbm.at[pl.ds(base, _RW)], idx_v)

        def gather(i, phase, sem):
            return pltpu.async_copy(
                table_hbm.at[idx_v.at[pl.ds(i * _CG, _CG)]],
                buf_v.at[phase], sem)

        gather(0, 0, sem0)

        def body(h, carry):
            i0 = 2 * h
            gather(i0 + 1, 1, sem1)
            pltpu.make_async_copy(
                table_hbm.at[idx_v.at[pl.ds(i0 * _CG, _CG)]],
                buf_v.at[0], sem0).wait()
            pltpu.sync_copy(buf_v.at[0],
                            out_hbm.at[pl.ds(base + i0 * _CG, _CG)])

            @pl.when(i0 + 2 < _NCHUNK)
            def _():
                gather(i0 + 2, 0, sem0)

            pltpu.make_async_copy(
                table_hbm.at[idx_v.at[pl.ds((i0 + 1) * _CG, _CG)]],
                buf_v.at[1], sem1).wait()
            pltpu.sync_copy(buf_v.at[1],
                            out_hbm.at[pl.ds(base + (i0 + 1) * _CG, _CG)])
            return carry

        lax.fori_loop(0, _NCHUNK // 2, body, 0)

    return k(table, idx_flat)


S3 = 512
T3 = S3 * K
_TPB = N // S3


def _mlp_body(g_ref, p_ref, w1_ref, b1_ref, w2_ref, b2_ref, w3_ref, b3_ref,
              out_ref):
    x = g_ref[...]
    a = p_ref[0]
    w1x = w1_ref[:, D:C_IN]
    corr = lax.dot_general(a, w1x, (((1,), (1,)), ((), ())),
                           preferred_element_type=jnp.float32)
    corr = jnp.reshape(
        jnp.broadcast_to(corr[:, None, :], (S3, K, C_IN)), (T3, C_IN))
    h = lax.dot_general(x, w1_ref[...], (((1,), (1,)), ((), ())),
                        preferred_element_type=jnp.float32)
    h = jnp.maximum(h - corr + b1_ref[...], 0.0)
    h = lax.dot_general(h, w2_ref[...], (((1,), (1,)), ((), ())),
                        preferred_element_type=jnp.float32)
    h = jnp.maximum(h + b2_ref[...], 0.0)
    y = lax.dot_general(w3_ref[...], h, (((1,), (1,)), ((), ())),
                        preferred_element_type=jnp.float32)
    out_ref[0] = y + b3_ref[...]


def _mlp(g, p, W1, b1, W2, b2, W3, b3):
    full = lambda shape: pl.BlockSpec(shape, lambda t: tuple(0 for _ in shape))
    out = pl.pallas_call(
        _mlp_body,
        grid=(B * N // S3,),
        in_specs=[
            pl.BlockSpec((T3, C_IN), lambda t: (t, 0)),
            pl.BlockSpec((1, S3, 3), lambda t: (t // _TPB, t % _TPB, 0)),
            full((C_IN, C_IN)),
            full((1, C_IN)),
            full((C_IN, C_IN)),
            full((1, C_IN)),
            full((C_OUT, C_IN)),
            full((C_OUT, 1)),
        ],
        out_specs=pl.BlockSpec((1, C_OUT, T3), lambda t: (t // _TPB, 0, t % _TPB)),
        out_shape=jax.ShapeDtypeStruct((B, C_OUT, N * K), jnp.float32),
    )(g, p, W1, b1.reshape(1, C_IN), W2, b2.reshape(1, C_IN), W3,
      b3.reshape(C_OUT, 1))
    return out.reshape(B, C_OUT, N, K)


def kernel(xyz, points, W1, b1, W2, b2, W3, b3):
    p = jnp.moveaxis(xyz, 1, 2)
    table = jnp.concatenate(
        [jnp.moveaxis(points, 1, 2), p], axis=-1).reshape(B * N, C_IN)
    idx = _knn(xyz, p)
    g = _gather_sc(table, idx.reshape(R))
    return _mlp(g, p, W1, b1, W2, b2, W3, b3)
